# baseline (speedup 1.0000x reference)
import functools

import jax
import jax.numpy as jnp
from jax import lax
from jax.experimental import pallas as pl
from jax.experimental.pallas import tpu as pltpu
from jax.experimental.pallas import tpu_sc as plsc

E = 8
K = 2
D = 1024
F = 2048
T = 512
CAP = 512
TS = 256
TPE = CAP // TS
FC = 512
NF = F // FC
NW = 32


def _router_body(x_ref, wg_ref, pos_ref, p_ref, cnt_ref):
    x = x_ref[...]
    wg = wg_ref[...]
    scores = jnp.dot(x, wg, preferred_element_type=jnp.float32)
    cols = lax.broadcasted_iota(jnp.int32, (T, E), 1)
    m1 = jnp.max(scores, axis=1, keepdims=True)
    e1 = jnp.min(jnp.where(scores == m1, cols, E), axis=1)
    neg = jnp.float32(-jnp.inf)
    sc2 = jnp.where(cols == e1[:, None], neg, scores)
    m2 = jnp.max(sc2, axis=1, keepdims=True)
    e2 = jnp.min(jnp.where(sc2 == m2, cols, E), axis=1)
    z = jnp.exp(m2[:, 0] - m1[:, 0])
    p1 = 1.0 / (1.0 + z)
    p2 = 1.0 - p1
    a = jnp.concatenate([e1, e2])
    p = jnp.concatenate([p1, p2])
    KT = K * T
    ecols = lax.broadcasted_iota(jnp.int32, (KT, E), 1)
    M = (a[:, None] == ecols).astype(jnp.float32)
    ri = lax.broadcasted_iota(jnp.int32, (KT, KT), 0)
    ci = lax.broadcasted_iota(jnp.int32, (KT, KT), 1)
    L = (ri > ci).astype(jnp.float32)
    R = jnp.dot(L, M, preferred_element_type=jnp.float32)
    rank = jnp.sum(M * R, axis=1).astype(jnp.int32)
    pos = a * CAP + rank
    pos_ref[...] = pos.reshape(8, 128)
    p_ref[...] = p.reshape(8, 128)
    cnt_ref[...] = jnp.sum(M, axis=0, keepdims=True).astype(jnp.int32)


def _router(xf, Wg):
    return pl.pallas_call(
        _router_body,
        out_shape=(
            jax.ShapeDtypeStruct((8, 128), jnp.int32),
            jax.ShapeDtypeStruct((8, 128), jnp.float32),
            jax.ShapeDtypeStruct((1, E), jnp.int32),
        ),
    )(xf, Wg)


def _dispatch(xf, pos32):
    mesh = plsc.VectorSubcoreMesh(core_axis_name="c", subcore_axis_name="s")

    @functools.partial(
        pl.kernel,
        out_type=jax.ShapeDtypeStruct((E * CAP, D), jnp.float32),
        mesh=mesh,
        scratch_types=[
            pltpu.VMEM((NW,), jnp.int32),
            pltpu.VMEM((NW, D), jnp.float32),
            pltpu.SemaphoreType.DMA,
        ],
    )
    def k(x_hbm, pos_hbm, xs_hbm, idx_v, rows_v, sem):
        wid = lax.axis_index("s") * 2 + lax.axis_index("c")
        src = (NW * wid) % T
        pltpu.sync_copy(pos_hbm.at[wid], idx_v)
        pltpu.sync_copy(x_hbm.at[pl.ds(src, NW)], rows_v)
        pltpu.async_copy(rows_v, xs_hbm.at[idx_v], sem).wait()

    return k(xf, pos32)


def _ffn_body(cnt_ref, xs_ref, w1_ref, w2_ref, w3_ref, out_ref):
    e = pl.program_id(0)
    t = pl.program_id(1)
    f = pl.program_id(2)
    active = cnt_ref[e] > t * TS

    @pl.when(active)
    def _():
        xt = xs_ref[...]
        w1 = w1_ref[0]
        w2 = w2_ref[0]
        w3 = w3_ref[0]
        g = jnp.dot(xt, w1, preferred_element_type=jnp.float32)
        u = jnp.dot(xt, w2, preferred_element_type=jnp.float32)
        h = (g * (1.0 / (1.0 + jnp.exp(-g)))) * u
        contrib = jnp.dot(h, w3, preferred_element_type=jnp.float32)

        @pl.when(f == 0)
        def _():
            out_ref[...] = contrib

        @pl.when(f != 0)
        def _():
            out_ref[...] += contrib


def _ffn(xs, W1, W2, W3, counts):
    def fw(e, t, f, cnt):
        return jnp.where(cnt[e] > t * TS, f, NF - 1)

    grid_spec = pltpu.PrefetchScalarGridSpec(
        num_scalar_prefetch=1,
        grid=(E, TPE, NF),
        in_specs=[
            pl.BlockSpec((TS, D), lambda e, t, f, cnt: (e * TPE + t, 0)),
            pl.BlockSpec((1, D, FC), lambda e, t, f, cnt: (e, 0, fw(e, t, f, cnt))),
            pl.BlockSpec((1, D, FC), lambda e, t, f, cnt: (e, 0, fw(e, t, f, cnt))),
            pl.BlockSpec((1, FC, D), lambda e, t, f, cnt: (e, fw(e, t, f, cnt), 0)),
        ],
        out_specs=pl.BlockSpec((TS, D), lambda e, t, f, cnt: (e * TPE + t, 0)),
    )
    return pl.pallas_call(
        _ffn_body,
        grid_spec=grid_spec,
        out_shape=jax.ShapeDtypeStruct((E * CAP, D), jnp.float32),
    )(counts, xs, W1, W2, W3)


def _combine(rows, posA, posB, pA, pB):
    mesh = plsc.VectorSubcoreMesh(core_axis_name="c", subcore_axis_name="s")
    TW = T // NW

    @functools.partial(
        pl.kernel,
        out_type=jax.ShapeDtypeStruct((T, D), jnp.float32),
        mesh=mesh,
        scratch_types=[
            pltpu.VMEM((TW,), jnp.int32),
            pltpu.VMEM((TW,), jnp.int32),
            pltpu.VMEM((TW, D), jnp.float32),
            pltpu.VMEM((TW, D), jnp.float32),
            pltpu.VMEM((TW, 16), jnp.float32),
            pltpu.VMEM((TW, 16), jnp.float32),
            pltpu.VMEM((TW, D), jnp.float32),
            pltpu.SemaphoreType.DMA,
            pltpu.SemaphoreType.DMA,
        ],
    )
    def k(rows_hbm, posA_hbm, posB_hbm, pA_hbm, pB_hbm, y_hbm,
          idxA, idxB, rA, rB, pAv, pBv, yv, semA, semB):
        wid = lax.axis_index("s") * 2 + lax.axis_index("c")
        t0 = TW * wid
        pltpu.sync_copy(posA_hbm.at[wid], idxA)
        pltpu.sync_copy(posB_hbm.at[wid], idxB)
        cA = pltpu.async_copy(rows_hbm.at[idxA], rA, semA)
        cB = pltpu.async_copy(rows_hbm.at[idxB], rB, semB)
        pltpu.sync_copy(pA_hbm.at[pl.ds(t0, TW)], pAv)
        pltpu.sync_copy(pB_hbm.at[pl.ds(t0, TW)], pBv)
        cA.wait()
        cB.wait()

        def body_j(j, _):
            a = pAv[j]
            b = pBv[j]
            for c in range(D // 16):
                s = pl.ds(c * 16, 16)
                yv[j, s] = a * rA[j, s] + b * rB[j, s]
            return 0

        lax.fori_loop(0, TW, body_j, 0)
        pltpu.sync_copy(yv, y_hbm.at[pl.ds(t0, TW)])

    return k(rows, posA, posB, pA, pB)


def kernel(x, Wg, W1, W2, W3):
    b, s, d = x.shape
    xf = x.reshape(b * s, d)
    pos8, p8, counts = _router(xf, Wg)
    pos = pos8.reshape(K * T)
    xs = _dispatch(xf, pos.reshape(NW, (K * T) // NW))
    rows = _ffn(xs, W1, W2, W3, counts.reshape(E))
    posA = pos[:T].reshape(NW, T // NW)
    posB = pos[T:].reshape(NW, T // NW)
    p = p8.reshape(K * T)
    pA = jnp.broadcast_to(p[:T, None], (T, 16))
    pB = jnp.broadcast_to(p[T:, None], (T, 16))
    y = _combine(rows, posA, posB, pA, pB)
    return y.reshape(b, s, d)


# speedup vs baseline: 1.5512x; 1.5512x over previous
import jax
import jax.numpy as jnp
from jax import lax
from jax.experimental import pallas as pl
from jax.experimental.pallas import tpu as pltpu

E = 8
K = 2
D = 1024
F = 2048
T = 512
KT = K * T
TS = 256
NTILES = 11
NS = NTILES * TS
FC = 512
NF = F // FC


def _router_body(x_ref, wg_ref, tok_ref, pb_ref, te_ref, va_ref):
    x = x_ref[...]
    wg = wg_ref[...]
    scores = jnp.dot(x, wg, preferred_element_type=jnp.float32)
    cols = lax.broadcasted_iota(jnp.int32, (T, E), 1)
    m1 = jnp.max(scores, axis=1, keepdims=True)
    e1 = jnp.min(jnp.where(scores == m1, cols, E), axis=1)
    neg = jnp.float32(-jnp.inf)
    sc2 = jnp.where(cols == e1[:, None], neg, scores)
    m2 = jnp.max(sc2, axis=1, keepdims=True)
    e2 = jnp.min(jnp.where(sc2 == m2, cols, E), axis=1)
    z = jnp.exp(m2[:, 0] - m1[:, 0])
    p1 = 1.0 / (1.0 + z)
    p2 = 1.0 - p1
    a = jnp.concatenate([e1, e2])
    p = jnp.concatenate([p1, p2])

    ecols = lax.broadcasted_iota(jnp.int32, (KT, E), 1)
    M = (a[:, None] == ecols).astype(jnp.float32)
    ri = lax.broadcasted_iota(jnp.int32, (KT, KT), 0)
    ci = lax.broadcasted_iota(jnp.int32, (KT, KT), 1)
    L = (ri > ci).astype(jnp.float32)
    R = jnp.dot(L, M, preferred_element_type=jnp.float32)
    rank = jnp.sum(M * R, axis=1)

    c = jnp.sum(M, axis=0)
    nt = jnp.ceil(c * (1.0 / TS))
    ei = lax.broadcasted_iota(jnp.int32, (E, E), 0)
    ej = lax.broadcasted_iota(jnp.int32, (E, E), 1)
    po = jnp.sum(jnp.where(ej < ei, (nt * TS)[None, :], 0.0), axis=1)
    po_end = po + nt * TS
    total = jnp.sum(nt) * TS

    po_a = jnp.sum(M * po[None, :], axis=1)
    pos = (po_a + rank).astype(jnp.int32)

    jj = lax.broadcasted_iota(jnp.int32, (KT, NS), 1)
    A = (pos[:, None] == jj).astype(jnp.float32)
    toki = lax.iota(jnp.int32, T).astype(jnp.float32)
    tokf = jnp.concatenate([toki, toki])
    tok_ref[...] = jnp.dot(tokf[None, :], A,
                           preferred_element_type=jnp.float32).astype(jnp.int32)
    pb_ref[...] = jnp.dot(p[None, :], A, preferred_element_type=jnp.float32)

    ti = lax.iota(jnp.int32, NTILES)
    starts = (ti * TS).astype(jnp.float32)
    te_raw = jnp.sum((po_end[None, :] <= starts[:, None]).astype(jnp.int32),
                     axis=1)
    eid = lax.iota(jnp.int32, E)
    last_e = jnp.max(jnp.where(c > 0, eid, -1))
    valid = starts < total
    te_ref[...] = jnp.where(valid, jnp.clip(te_raw, 0, E - 1), last_e)[None, :]
    va_ref[...] = valid.astype(jnp.int32)[None, :]


def _router(xf, Wg):
    return pl.pallas_call(
        _router_body,
        out_shape=(
            jax.ShapeDtypeStruct((1, NS), jnp.int32),
            jax.ShapeDtypeStruct((1, NS), jnp.float32),
            jax.ShapeDtypeStruct((1, NTILES), jnp.int32),
            jax.ShapeDtypeStruct((1, NTILES), jnp.int32),
        ),
    )(xf, Wg)


def _mega_body(te_ref, va_ref, x_ref, w1_ref, w2_ref, w3_ref, tok_ref, pb_ref,
               y_ref, xs_sc, acc_sc):
    i = pl.program_id(0)
    f = pl.program_id(1)

    @pl.when(jnp.logical_and(i == 0, f == 0))
    def _():
        y_ref[...] = jnp.zeros_like(y_ref)

    @pl.when(va_ref[i] != 0)
    def _():
        tok = tok_ref[0, 0]

        @pl.when(f == 0)
        def _():
            tcols = lax.broadcasted_iota(jnp.int32, (TS, T), 1)
            oh = (tok[:, None] == tcols).astype(jnp.float32)
            xs_sc[...] = jnp.dot(oh, x_ref[...],
                                 preferred_element_type=jnp.float32)

        xt = xs_sc[...]
        w1 = w1_ref[0]
        w2 = w2_ref[0]
        w3 = w3_ref[0]
        g = jnp.dot(xt, w1, preferred_element_type=jnp.float32)
        u = jnp.dot(xt, w2, preferred_element_type=jnp.float32)
        contrib = jnp.dot((g * (1.0 / (1.0 + jnp.exp(-g)))) * u, w3,
                          preferred_element_type=jnp.float32)

        @pl.when(f == 0)
        def _():
            acc_sc[...] = contrib

        @pl.when(f != 0)
        def _():
            acc_sc[...] += contrib

        @pl.when(f == NF - 1)
        def _():
            trows = lax.broadcasted_iota(jnp.int32, (T, TS), 0)
            c2 = jnp.where(tok[None, :] == trows, pb_ref[0, 0][None, :], 0.0)
            y_ref[...] += jnp.dot(c2, acc_sc[...],
                                  preferred_element_type=jnp.float32)


def _mega(xf, W1, W2, W3, tok_slot, pb_slot, te, valid):
    def fe(i, f, te_r, va_r):
        return te_r[i]

    def fw(i, f, te_r, va_r):
        return jnp.where(va_r[i] != 0, f, NF - 1)

    grid_spec = pltpu.PrefetchScalarGridSpec(
        num_scalar_prefetch=2,
        grid=(NTILES, NF),
        in_specs=[
            pl.BlockSpec((T, D), lambda i, f, te_r, va_r: (0, 0)),
            pl.BlockSpec((1, D, FC), lambda i, f, te_r, va_r:
                         (fe(i, f, te_r, va_r), 0, fw(i, f, te_r, va_r))),
            pl.BlockSpec((1, D, FC), lambda i, f, te_r, va_r:
                         (fe(i, f, te_r, va_r), 0, fw(i, f, te_r, va_r))),
            pl.BlockSpec((1, FC, D), lambda i, f, te_r, va_r:
                         (fe(i, f, te_r, va_r), fw(i, f, te_r, va_r), 0)),
            pl.BlockSpec((1, 1, TS), lambda i, f, te_r, va_r: (i, 0, 0)),
            pl.BlockSpec((1, 1, TS), lambda i, f, te_r, va_r: (i, 0, 0)),
        ],
        out_specs=pl.BlockSpec((T, D), lambda i, f, te_r, va_r: (0, 0)),
        scratch_shapes=[
            pltpu.VMEM((TS, D), jnp.float32),
            pltpu.VMEM((TS, D), jnp.float32),
        ],
    )
    return pl.pallas_call(
        _mega_body,
        grid_spec=grid_spec,
        out_shape=jax.ShapeDtypeStruct((T, D), jnp.float32),
    )(te, valid, xf, W1, W2, W3, tok_slot, pb_slot)


def kernel(x, Wg, W1, W2, W3):
    b, s, d = x.shape
    xf = x.reshape(b * s, d)
    tok_slot, pb_slot, te, valid = _router(xf, Wg)
    y = _mega(
        xf, W1, W2, W3,
        tok_slot.reshape(NTILES, 1, TS),
        pb_slot.reshape(NTILES, 1, TS),
        te.reshape(NTILES),
        valid.reshape(NTILES),
    )
    return y.reshape(b, s, d)
